# Initial kernel scaffold; baseline (speedup 1.0000x reference)
import jax
import jax.numpy as jnp
from jax.experimental import pallas as pl


def kernel(x_o, x_a, edge_index, e_type, e_type1, idx, W1, root1, b1, W2, root2, b2, attt, Wd, bd, features1, L1, bl1, L2, bl2, L3, bl3):
    raise NotImplementedError("write your pallas kernel here")


# speedup vs baseline: 4.9826x; 4.9826x over previous
import functools

import jax
import jax.numpy as jnp
from jax import lax
from jax.experimental import pallas as pl
from jax.experimental.pallas import tpu as pltpu
from jax.experimental.pallas import tpu_sc as plsc

_NC = 2
_NS = 16
_NW = _NC * _NS
_CH = 128

_f32 = jnp.float32
_i32 = jnp.int32


def _mesh():
    return plsc.VectorSubcoreMesh(core_axis_name="c", subcore_axis_name="s")


_SC_PARAMS = pltpu.CompilerParams(use_tc_tiling_on_sc=False,
                                 needs_layout_passes=False)


def _prep(src, dst, et0, et1, n, r):
    e = src.shape[0]
    nch = e // _CH
    nr = n * r
    nrp = ((nr + _NS * 256 - 1) // (_NS * 256)) * (_NS * 256)
    ps = nrp // _NS
    t1 = -(-nch // _NS)
    t2 = -(-nch // _NW)

    def body(src_h, dst_h, et0_h, et1_h, zc_h,
             skey0_h, skey1_h, w0_h, w1_h,
             src_v, dst_v, t0_v, t1_v, dk0_v, dk1_v, sk0_v, sk1_v,
             ones_v, c_v, w_v, cnt0_sh, cnt1_sh):
        cid = lax.axis_index("c")
        sid = lax.axis_index("s")
        wid = sid * _NC + cid
        z0 = sid * ps
        pltpu.sync_copy(zc_h.at[pl.ds(z0, ps)], cnt0_sh.at[pl.ds(z0, ps)])
        pltpu.sync_copy(zc_h.at[pl.ds(z0, ps)], cnt1_sh.at[pl.ds(z0, ps)])
        for j in range(_CH // 16):
            ones_v[0, pl.ds(j * 16, 16)] = jnp.ones((16,), _f32)
        plsc.subcore_barrier()

        @pl.loop(0, t1)
        def _(i):
            chunk = sid + i * _NS

            @pl.when(chunk < nch)
            def _():
                base = chunk * _CH
                pltpu.sync_copy(dst_h.at[pl.ds(base, _CH)], dst_v.at[0])
                pltpu.sync_copy(et0_h.at[pl.ds(base, _CH)], t0_v.at[0])
                pltpu.sync_copy(et1_h.at[pl.ds(base, _CH)], t1_v.at[0])
                for j in range(_CH // 16):
                    sl = (0, pl.ds(j * 16, 16))
                    d = dst_v[sl]
                    dk0_v[sl] = d * r + t0_v[sl]
                    dk1_v[sl] = d * r + t1_v[sl]
                pltpu.sync_copy(ones_v.at[0], cnt0_sh.at[dk0_v.at[0]], add=True)
                pltpu.sync_copy(ones_v.at[0], cnt1_sh.at[dk1_v.at[0]], add=True)

        plsc.subcore_barrier()

        @pl.loop(0, t2)
        def _(i):
            chunk = wid + i * _NW

            @pl.when(chunk < nch)
            def _():
                base = chunk * _CH
                pltpu.sync_copy(src_h.at[pl.ds(base, _CH)], src_v.at[0])
                pltpu.sync_copy(dst_h.at[pl.ds(base, _CH)], dst_v.at[0])
                pltpu.sync_copy(et0_h.at[pl.ds(base, _CH)], t0_v.at[0])
                pltpu.sync_copy(et1_h.at[pl.ds(base, _CH)], t1_v.at[0])
                for j in range(_CH // 16):
                    sl = (0, pl.ds(j * 16, 16))
                    s = src_v[sl]
                    d = dst_v[sl]
                    sk0_v[sl] = s * r + t0_v[sl]
                    sk1_v[sl] = s * r + t1_v[sl]
                    dk0_v[sl] = d * r + t0_v[sl]
                    dk1_v[sl] = d * r + t1_v[sl]
                pltpu.sync_copy(sk0_v.at[0], skey0_h.at[pl.ds(base, _CH)])
                pltpu.sync_copy(sk1_v.at[0], skey1_h.at[pl.ds(base, _CH)])
                pltpu.sync_copy(cnt0_sh.at[dk0_v.at[0]], c_v.at[0])
                for j in range(_CH // 16):
                    sl = (0, pl.ds(j * 16, 16))
                    w_v[sl] = 1.0 / c_v[sl]
                pltpu.sync_copy(w_v.at[0], w0_h.at[pl.ds(base, _CH)])
                pltpu.sync_copy(cnt1_sh.at[dk1_v.at[0]], c_v.at[0])
                for j in range(_CH // 16):
                    sl = (0, pl.ds(j * 16, 16))
                    w_v[sl] = 1.0 / c_v[sl]
                pltpu.sync_copy(w_v.at[0], w1_h.at[pl.ds(base, _CH)])

    kern = pl.kernel(
        body,
        out_type=[jax.ShapeDtypeStruct((e,), _i32),
                  jax.ShapeDtypeStruct((e,), _i32),
                  jax.ShapeDtypeStruct((e,), _f32),
                  jax.ShapeDtypeStruct((e,), _f32)],
        mesh=_mesh(),
        compiler_params=_SC_PARAMS,
        scratch_types=[
            pltpu.VMEM((1, _CH), _i32),
            pltpu.VMEM((1, _CH), _i32),
            pltpu.VMEM((1, _CH), _i32),
            pltpu.VMEM((1, _CH), _i32),
            pltpu.VMEM((1, _CH), _i32),
            pltpu.VMEM((1, _CH), _i32),
            pltpu.VMEM((1, _CH), _i32),
            pltpu.VMEM((1, _CH), _i32),
            pltpu.VMEM((1, _CH), _f32),
            pltpu.VMEM((1, _CH), _f32),
            pltpu.VMEM((1, _CH), _f32),
            pltpu.VMEM_SHARED((nrp,), _f32),
            pltpu.VMEM_SHARED((nrp,), _f32),
        ],
    )
    zc = jnp.zeros((nrp,), _f32)
    return kern(src, dst, et0, et1, zc)


def _conv_pair(tabx, taby, skey, w, dst, n, h):
    e = skey.shape[0]
    nch = e // _CH
    t = -(-nch // _NW)
    psn = n // _NS

    def body(tabx_h, taby_h, sk_h, w_h, dst_h, zn_h, ax_h, ay_h,
             sk_v, dst_v, w_v, rx_v, ry_v, ax_sh, ay_sh):
        cid = lax.axis_index("c")
        sid = lax.axis_index("s")
        wid = sid * _NC + cid
        r0 = sid * psn
        pltpu.sync_copy(zn_h.at[pl.ds(r0, psn)], ax_sh.at[pl.ds(r0, psn)])
        pltpu.sync_copy(zn_h.at[pl.ds(r0, psn)], ay_sh.at[pl.ds(r0, psn)])
        plsc.subcore_barrier()

        @pl.loop(0, t)
        def _(i):
            chunk = wid + i * _NW

            @pl.when(chunk < nch)
            def _():
                base = chunk * _CH
                pltpu.sync_copy(sk_h.at[pl.ds(base, _CH)], sk_v.at[0])
                pltpu.sync_copy(w_h.at[pl.ds(base, _CH)], w_v.at[0])
                pltpu.sync_copy(dst_h.at[pl.ds(base, _CH)], dst_v.at[0])
                pltpu.sync_copy(tabx_h.at[sk_v.at[0]], rx_v)
                pltpu.sync_copy(taby_h.at[sk_v.at[0]], ry_v)

                @pl.loop(0, _CH)
                def _(ei):
                    idxv = jnp.full((16,), ei, _i32)
                    wb = plsc.load_gather(w_v.at[0], [idxv])
                    for j in range(h // 16):
                        sl = (ei, pl.ds(j * 16, 16))
                        rx_v[sl] = rx_v[sl] * wb
                        ry_v[sl] = ry_v[sl] * wb

                pltpu.sync_copy(rx_v, ax_sh.at[dst_v.at[0]], add=True)
                pltpu.sync_copy(ry_v, ay_sh.at[dst_v.at[0]], add=True)

        plsc.subcore_barrier()
        pltpu.sync_copy(ax_sh.at[pl.ds(r0, psn)], ax_h.at[cid, pl.ds(r0, psn)])
        pltpu.sync_copy(ay_sh.at[pl.ds(r0, psn)], ay_h.at[cid, pl.ds(r0, psn)])

    kern = pl.kernel(
        body,
        out_type=[jax.ShapeDtypeStruct((_NC, n, h), _f32),
                  jax.ShapeDtypeStruct((_NC, n, h), _f32)],
        mesh=_mesh(),
        compiler_params=_SC_PARAMS,
        scratch_types=[
            pltpu.VMEM((1, _CH), _i32),
            pltpu.VMEM((1, _CH), _i32),
            pltpu.VMEM((1, _CH), _f32),
            pltpu.VMEM((_CH, h), _f32),
            pltpu.VMEM((_CH, h), _f32),
            pltpu.VMEM_SHARED((n, h), _f32),
            pltpu.VMEM_SHARED((n, h), _f32),
        ],
    )
    zn = jnp.zeros((n, h), _f32)
    return kern(tabx, taby, skey, w, dst, zn)


def _pairs(fin, feat, aa, bb, b):
    n, fd = fin.shape
    fr = feat.shape[1]
    nch = b // _CH

    def body(fin_h, feat_h, aa_h, bb_h, ddi_h, mol_h,
             ia_v, ib_v, fa_v, fb_v, ma_v, mb_v):
        cid = lax.axis_index("c")
        sid = lax.axis_index("s")
        wid = sid * _NC + cid

        @pl.loop(0, -(-nch // _NW))
        def _(i):
            chunk = wid + i * _NW

            @pl.when(chunk < nch)
            def _():
                base = chunk * _CH
                pltpu.sync_copy(aa_h.at[pl.ds(base, _CH)], ia_v.at[0])
                pltpu.sync_copy(bb_h.at[pl.ds(base, _CH)], ib_v.at[0])
                pltpu.sync_copy(fin_h.at[ia_v.at[0]], fa_v)
                pltpu.sync_copy(fin_h.at[ib_v.at[0]], fb_v)
                pltpu.sync_copy(feat_h.at[ia_v.at[0]], ma_v)
                pltpu.sync_copy(feat_h.at[ib_v.at[0]], mb_v)
                pltpu.sync_copy(fa_v, ddi_h.at[pl.ds(base, _CH), pl.ds(0, fd)])
                pltpu.sync_copy(fb_v, ddi_h.at[pl.ds(base, _CH), pl.ds(fd, fd)])
                pltpu.sync_copy(ma_v, mol_h.at[pl.ds(base, _CH), pl.ds(0, fr)])
                pltpu.sync_copy(mb_v, mol_h.at[pl.ds(base, _CH), pl.ds(fr, fr)])

    kern = pl.kernel(
        body,
        out_type=[jax.ShapeDtypeStruct((b, 2 * fd), _f32),
                  jax.ShapeDtypeStruct((b, 2 * fr), _f32)],
        mesh=_mesh(),
        compiler_params=_SC_PARAMS,
        scratch_types=[
            pltpu.VMEM((1, _CH), _i32),
            pltpu.VMEM((1, _CH), _i32),
            pltpu.VMEM((_CH, fd), _f32),
            pltpu.VMEM((_CH, fd), _f32),
            pltpu.VMEM((_CH, fr), _f32),
            pltpu.VMEM((_CH, fr), _f32),
        ],
    )
    return kern(fin, feat, aa, bb)


_PREC = lax.Precision.HIGHEST


def _dot(a, b):
    return jnp.dot(a, b, preferred_element_type=_f32, precision=_PREC)


def _mm_tab(x, wt, wr, br):
    n, k = x.shape
    ct = wt.shape[1]
    cr = wr.shape[1]

    def body(x_ref, wt_ref, wr_ref, tab_ref, rt_ref):
        xb = x_ref[...]
        tab_ref[...] = _dot(xb, wt_ref[...])
        rt_ref[...] = _dot(xb, wr_ref[...])

    return pl.pallas_call(
        body,
        grid=(n // br,),
        in_specs=[pl.BlockSpec((br, k), lambda i: (i, 0)),
                  pl.BlockSpec((k, ct), lambda i: (0, 0)),
                  pl.BlockSpec((k, cr), lambda i: (0, 0))],
        out_specs=[pl.BlockSpec((br, ct), lambda i: (i, 0)),
                   pl.BlockSpec((br, cr), lambda i: (i, 0))],
        out_shape=[jax.ShapeDtypeStruct((n, ct), _f32),
                   jax.ShapeDtypeStruct((n, cr), _f32)],
    )(x, wt, wr)


def _combine_mm(accp, rt, bvec, wt, wr, br):
    n, h = rt.shape
    ct = wt.shape[1]
    cr = wr.shape[1]

    def body(acc_ref, rt_ref, b_ref, wt_ref, wr_ref, x1_ref, tab_ref, rt2_ref):
        x1 = acc_ref[0] + acc_ref[1] + rt_ref[...] + b_ref[...]
        x1 = jnp.maximum(x1, 0.0)
        x1_ref[...] = x1
        tab_ref[...] = _dot(x1, wt_ref[...])
        rt2_ref[...] = _dot(x1, wr_ref[...])

    return pl.pallas_call(
        body,
        grid=(n // br,),
        in_specs=[pl.BlockSpec((_NC, br, h), lambda i: (0, i, 0)),
                  pl.BlockSpec((br, h), lambda i: (i, 0)),
                  pl.BlockSpec((1, h), lambda i: (0, 0)),
                  pl.BlockSpec((h, ct), lambda i: (0, 0)),
                  pl.BlockSpec((h, cr), lambda i: (0, 0))],
        out_specs=[pl.BlockSpec((br, h), lambda i: (i, 0)),
                   pl.BlockSpec((br, ct), lambda i: (i, 0)),
                   pl.BlockSpec((br, cr), lambda i: (i, 0))],
        out_shape=[jax.ShapeDtypeStruct((n, h), _f32),
                   jax.ShapeDtypeStruct((n, ct), _f32),
                   jax.ShapeDtypeStruct((n, cr), _f32)],
    )(accp, rt, bvec, wt, wr)


def _x2_combine(accp, rt2, bvec, br):
    n, h = rt2.shape
    nb = n // br

    def body(acc_ref, rt_ref, b_ref, x2_ref, ps_ref):
        x2 = acc_ref[0] + acc_ref[1] + rt_ref[...] + b_ref[...]
        x2_ref[...] = x2
        ps_ref[...] = jnp.sum(x2, axis=0, keepdims=True)[None]

    x2, ps = pl.pallas_call(
        body,
        grid=(nb,),
        in_specs=[pl.BlockSpec((_NC, br, h), lambda i: (0, i, 0)),
                  pl.BlockSpec((br, h), lambda i: (i, 0)),
                  pl.BlockSpec((1, h), lambda i: (0, 0))],
        out_specs=[pl.BlockSpec((br, h), lambda i: (i, 0)),
                   pl.BlockSpec((1, 1, h), lambda i: (i, 0, 0))],
        out_shape=[jax.ShapeDtypeStruct((n, h), _f32),
                   jax.ShapeDtypeStruct((nb, 1, h), _f32)],
    )(accp, rt2, bvec)
    return x2, ps.reshape(nb, h)


def _readout(psum, wd2t, n):
    nb, h = psum.shape

    def body(ps_ref, wd_ref, v_ref):
        tot = jnp.sum(ps_ref[...], axis=0, keepdims=True) * (1.0 / n)
        hvec = jax.nn.sigmoid(tot)
        v_ref[...] = _dot(hvec, wd_ref[...])

    return pl.pallas_call(
        body,
        grid=(1,),
        in_specs=[pl.BlockSpec((nb, h), lambda i: (0, 0)),
                  pl.BlockSpec((h, h), lambda i: (0, 0))],
        out_specs=[pl.BlockSpec((1, h), lambda i: (0, 0))],
        out_shape=[jax.ShapeDtypeStruct((1, h), _f32)],
    )(psum, wd2t)[0]


def _heads(x1o, x2o, x2oa, x2oaa, x2aa, v, bd, a0, a1, br):
    n, h1 = x1o.shape
    h2 = x2o.shape[1]

    def body(x1_ref, xo_ref, xoa_ref, xoaa_ref, xaa_ref, v_ref, bd_ref,
             a0_ref, a1_ref, ros_ref, roa_ref, rosa_ref, fin_ref):
        vv = v_ref[...]
        bdv = bd_ref[...]

        def mv(x):
            return jnp.sum(x * vv, axis=1, keepdims=True) + bdv

        xo = xo_ref[...]
        s_o = mv(xo)
        s_oa = mv(xoa_ref[...])
        s_oaa = mv(xoaa_ref[...])
        s_aa = mv(xaa_ref[...])
        ros_ref[...] = jnp.concatenate([s_o, s_oaa], axis=1)
        roa_ref[...] = jnp.concatenate([s_o, s_oa], axis=1)
        rosa_ref[...] = jnp.concatenate([s_o, s_aa], axis=1)
        fin_ref[...] = jnp.concatenate(
            [a0_ref[...] * x1_ref[...], a1_ref[...] * xo], axis=1)

    return pl.pallas_call(
        body,
        grid=(n // br,),
        in_specs=[pl.BlockSpec((br, h1), lambda i: (i, 0)),
                  pl.BlockSpec((br, h2), lambda i: (i, 0)),
                  pl.BlockSpec((br, h2), lambda i: (i, 0)),
                  pl.BlockSpec((br, h2), lambda i: (i, 0)),
                  pl.BlockSpec((br, h2), lambda i: (i, 0)),
                  pl.BlockSpec((1, h2), lambda i: (0, 0)),
                  pl.BlockSpec((1, 1), lambda i: (0, 0)),
                  pl.BlockSpec((1, 1), lambda i: (0, 0)),
                  pl.BlockSpec((1, 1), lambda i: (0, 0))],
        out_specs=[pl.BlockSpec((br, 2), lambda i: (i, 0)),
                   pl.BlockSpec((br, 2), lambda i: (i, 0)),
                   pl.BlockSpec((br, 2), lambda i: (i, 0)),
                   pl.BlockSpec((br, h1 + h2), lambda i: (i, 0))],
        out_shape=[jax.ShapeDtypeStruct((n, 2), _f32),
                   jax.ShapeDtypeStruct((n, 2), _f32),
                   jax.ShapeDtypeStruct((n, 2), _f32),
                   jax.ShapeDtypeStruct((n, h1 + h2), _f32)],
    )(x1o, x2o, x2oa, x2oaa, x2aa, v, bd, a0, a1)


def _mlp(ddi, l1, b1, l2, b2, l3, b3, br):
    b, d = ddi.shape
    c1 = l1.shape[1]
    c2 = l2.shape[1]
    c3 = l3.shape[1]

    def body(x_ref, l1_ref, b1_ref, l2_ref, b2_ref, l3_ref, b3_ref, o_ref):
        h = _dot(x_ref[...], l1_ref[...]) + b1_ref[...]
        h = jnp.where(h > 0, h, jnp.exp(h) - 1.0)
        h = _dot(h, l2_ref[...]) + b2_ref[...]
        h = jnp.where(h > 0, h, jnp.exp(h) - 1.0)
        o_ref[...] = _dot(h, l3_ref[...]) + b3_ref[...]

    return pl.pallas_call(
        body,
        grid=(b // br,),
        in_specs=[pl.BlockSpec((br, d), lambda i: (i, 0)),
                  pl.BlockSpec((d, c1), lambda i: (0, 0)),
                  pl.BlockSpec((1, c1), lambda i: (0, 0)),
                  pl.BlockSpec((c1, c2), lambda i: (0, 0)),
                  pl.BlockSpec((1, c2), lambda i: (0, 0)),
                  pl.BlockSpec((c2, c3), lambda i: (0, 0)),
                  pl.BlockSpec((1, c3), lambda i: (0, 0))],
        out_specs=[pl.BlockSpec((br, c3), lambda i: (i, 0))],
        out_shape=[jax.ShapeDtypeStruct((b, c3), _f32)],
    )(ddi, l1, b1, l2, b2, l3, b3)[0]


def kernel(x_o, x_a, edge_index, e_type, e_type1, idx, W1, root1, b1,
           W2, root2, b2, attt, Wd, bd, features1, L1, bl1, L2, bl2, L3, bl3):
    n, df = x_o.shape
    e = e_type.shape[0]
    r = W1.shape[0]
    h1 = W1.shape[2]
    h2 = W2.shape[2]
    b = idx.shape[1]
    src = edge_index[0]
    dst = edge_index[1]

    skey0, skey1, w0, w1 = _prep(src, dst, e_type, e_type1, n, r)

    wt1 = jnp.transpose(W1, (1, 0, 2)).reshape(df, r * h1)
    tab_o, rt_o = _mm_tab(x_o, wt1, root1, 1000)
    tab_a, rt_a = _mm_tab(x_a, wt1, root1, 1000)

    tabv_o = tab_o.reshape(n * r, h1)
    tabv_a = tab_a.reshape(n * r, h1)
    acc_o0, acc_a0 = _conv_pair(tabv_o, tabv_a, skey0, w0, dst, n, h1)
    acc_o1, acc_a1 = _conv_pair(tabv_o, tabv_a, skey1, w1, dst, n, h1)

    wt2 = jnp.transpose(W2, (1, 0, 2)).reshape(h1, r * h2)
    b1r = b1.reshape(1, h1)
    x1_o, tab2_o, rt2_o = _combine_mm(acc_o0, rt_o, b1r, wt2, root2, 1000)
    x1_oa, tab2_oa, rt2_oa = _combine_mm(acc_a0, rt_a, b1r, wt2, root2, 1000)
    x1_oaa, tab2_oaa, rt2_oaa = _combine_mm(acc_o1, rt_o, b1r, wt2, root2, 1000)
    x1_aa, tab2_aa, rt2_aa = _combine_mm(acc_a1, rt_a, b1r, wt2, root2, 1000)

    acc2_o, acc2_oa = _conv_pair(tab2_o.reshape(n * r, h2),
                                 tab2_oa.reshape(n * r, h2),
                                 skey0, w0, dst, n, h2)
    acc2_oaa, acc2_aa = _conv_pair(tab2_oaa.reshape(n * r, h2),
                                   tab2_aa.reshape(n * r, h2),
                                   skey1, w1, dst, n, h2)

    b2r = b2.reshape(1, h2)
    x2_o, ps_o = _x2_combine(acc2_o, rt2_o, b2r, 1000)
    x2_oa, _ = _x2_combine(acc2_oa, rt2_oa, b2r, 1000)
    x2_oaa, _ = _x2_combine(acc2_oaa, rt2_oaa, b2r, 1000)
    x2_aa, _ = _x2_combine(acc2_aa, rt2_aa, b2r, 1000)

    v = _readout(ps_o, jnp.transpose(Wd[0]), n)
    ret_os, ret_oa, ret_os_a, final = _heads(
        x1_o, x2_o, x2_oa, x2_oaa, x2_aa, v,
        bd.reshape(1, 1), attt[0].reshape(1, 1), attt[1].reshape(1, 1), 1000)

    final_DDI, final_molecule = _pairs(final, features1, idx[0], idx[1], b)

    log = _mlp(final_DDI, L1, bl1.reshape(1, -1), L2, bl2.reshape(1, -1),
               L3, bl3.reshape(1, -1), 1024)

    return (log, ret_os, ret_oa, ret_os_a, x2_o, final_DDI, final_molecule,
            x2_oaa, x2_aa)


# speedup vs baseline: 5.3403x; 1.0718x over previous
import functools

import jax
import jax.numpy as jnp
from jax import lax
from jax.experimental import pallas as pl
from jax.experimental.pallas import tpu as pltpu
from jax.experimental.pallas import tpu_sc as plsc

_NC = 2
_NS = 16
_NW = _NC * _NS
_CH = 128

_f32 = jnp.float32
_i32 = jnp.int32


def _mesh():
    return plsc.VectorSubcoreMesh(core_axis_name="c", subcore_axis_name="s")


_SC_PARAMS = pltpu.CompilerParams(use_tc_tiling_on_sc=False,
                                 needs_layout_passes=False)


def _prep(src, dst, et0, et1, n, r):
    e = src.shape[0]
    nch = e // _CH
    nr = n * r
    nrp = ((nr + _NS * 256 - 1) // (_NS * 256)) * (_NS * 256)
    ps = nrp // _NS
    t1 = -(-nch // _NS)
    t2 = -(-nch // _NW)

    def body(src_h, dst_h, et0_h, et1_h, zc_h,
             skey0_h, skey1_h, w0_h, w1_h,
             src_v, dst_v, t0_v, t1_v, dk0_v, dk1_v, sk0_v, sk1_v,
             ones_v, c_v, w_v, cnt0_sh, cnt1_sh):
        cid = lax.axis_index("c")
        sid = lax.axis_index("s")
        wid = sid * _NC + cid
        z0 = sid * ps
        pltpu.sync_copy(zc_h.at[pl.ds(z0, ps)], cnt0_sh.at[pl.ds(z0, ps)])
        pltpu.sync_copy(zc_h.at[pl.ds(z0, ps)], cnt1_sh.at[pl.ds(z0, ps)])
        for j in range(_CH // 16):
            ones_v[0, pl.ds(j * 16, 16)] = jnp.ones((16,), _f32)
        plsc.subcore_barrier()

        @pl.loop(0, t1)
        def _(i):
            chunk = sid + i * _NS

            @pl.when(chunk < nch)
            def _():
                base = chunk * _CH
                pltpu.sync_copy(dst_h.at[pl.ds(base, _CH)], dst_v.at[0])
                pltpu.sync_copy(et0_h.at[pl.ds(base, _CH)], t0_v.at[0])
                pltpu.sync_copy(et1_h.at[pl.ds(base, _CH)], t1_v.at[0])
                for j in range(_CH // 16):
                    sl = (0, pl.ds(j * 16, 16))
                    d = dst_v[sl]
                    dk0_v[sl] = d * r + t0_v[sl]
                    dk1_v[sl] = d * r + t1_v[sl]
                pltpu.sync_copy(ones_v.at[0], cnt0_sh.at[dk0_v.at[0]], add=True)
                pltpu.sync_copy(ones_v.at[0], cnt1_sh.at[dk1_v.at[0]], add=True)

        plsc.subcore_barrier()

        @pl.loop(0, t2)
        def _(i):
            chunk = wid + i * _NW

            @pl.when(chunk < nch)
            def _():
                base = chunk * _CH
                pltpu.sync_copy(src_h.at[pl.ds(base, _CH)], src_v.at[0])
                pltpu.sync_copy(dst_h.at[pl.ds(base, _CH)], dst_v.at[0])
                pltpu.sync_copy(et0_h.at[pl.ds(base, _CH)], t0_v.at[0])
                pltpu.sync_copy(et1_h.at[pl.ds(base, _CH)], t1_v.at[0])
                for j in range(_CH // 16):
                    sl = (0, pl.ds(j * 16, 16))
                    s = src_v[sl]
                    d = dst_v[sl]
                    sk0_v[sl] = s * r + t0_v[sl]
                    sk1_v[sl] = s * r + t1_v[sl]
                    dk0_v[sl] = d * r + t0_v[sl]
                    dk1_v[sl] = d * r + t1_v[sl]
                pltpu.sync_copy(sk0_v.at[0], skey0_h.at[pl.ds(base, _CH)])
                pltpu.sync_copy(sk1_v.at[0], skey1_h.at[pl.ds(base, _CH)])
                pltpu.sync_copy(cnt0_sh.at[dk0_v.at[0]], c_v.at[0])
                for j in range(_CH // 16):
                    sl = (0, pl.ds(j * 16, 16))
                    w_v[sl] = 1.0 / c_v[sl]
                pltpu.sync_copy(w_v.at[0], w0_h.at[pl.ds(base, _CH)])
                pltpu.sync_copy(cnt1_sh.at[dk1_v.at[0]], c_v.at[0])
                for j in range(_CH // 16):
                    sl = (0, pl.ds(j * 16, 16))
                    w_v[sl] = 1.0 / c_v[sl]
                pltpu.sync_copy(w_v.at[0], w1_h.at[pl.ds(base, _CH)])

    kern = pl.kernel(
        body,
        out_type=[jax.ShapeDtypeStruct((e,), _i32),
                  jax.ShapeDtypeStruct((e,), _i32),
                  jax.ShapeDtypeStruct((e,), _f32),
                  jax.ShapeDtypeStruct((e,), _f32)],
        mesh=_mesh(),
        compiler_params=_SC_PARAMS,
        scratch_types=[
            pltpu.VMEM((1, _CH), _i32),
            pltpu.VMEM((1, _CH), _i32),
            pltpu.VMEM((1, _CH), _i32),
            pltpu.VMEM((1, _CH), _i32),
            pltpu.VMEM((1, _CH), _i32),
            pltpu.VMEM((1, _CH), _i32),
            pltpu.VMEM((1, _CH), _i32),
            pltpu.VMEM((1, _CH), _i32),
            pltpu.VMEM((1, _CH), _f32),
            pltpu.VMEM((1, _CH), _f32),
            pltpu.VMEM((1, _CH), _f32),
            pltpu.VMEM_SHARED((nrp,), _f32),
            pltpu.VMEM_SHARED((nrp,), _f32),
        ],
    )
    zc = jnp.zeros((nrp,), _f32)
    return kern(src, dst, et0, et1, zc)


_CPW = 84


def _conv_pair(tabx, taby, sk2, w2, dst2, n, h, s):
    nchp = sk2.shape[0]
    assert nchp == _NW * _CPW
    ng = _CPW // s
    assert ng % 3 == 0
    psn = n // _NS

    def body(tabx_h, taby_h, sk_h, w_h, dst_h, zn_h, ax_h, ay_h,
             sk_v, dst_v, w_v, rx_v, ry_v, ax_sh, ay_sh,
             sg0, sg1, sg2, ss0, ss1, ss2):
        cid = lax.axis_index("c")
        sid = lax.axis_index("s")
        wid = sid * _NC + cid
        r0 = sid * psn
        pltpu.sync_copy(zn_h.at[pl.ds(r0, psn)], ax_sh.at[pl.ds(r0, psn)])
        pltpu.sync_copy(zn_h.at[pl.ds(r0, psn)], ay_sh.at[pl.ds(r0, psn)])
        plsc.subcore_barrier()
        cw = wid * _CPW
        sg = (sg0, sg1, sg2)
        ss = (ss0, ss1, ss2)

        def load_group(g, b):
            c0 = cw + g * s
            pltpu.sync_copy(sk_h.at[pl.ds(c0, s)], sk_v.at[pl.ds(b * s, s)])
            pltpu.sync_copy(w_h.at[pl.ds(c0, s)], w_v.at[pl.ds(b * s, s)])
            pltpu.sync_copy(dst_h.at[pl.ds(c0, s)], dst_v.at[pl.ds(b * s, s)])

        def start_gathers(b):
            for k in range(s):
                row = (b * s + k) * _CH
                pltpu.async_copy(tabx_h.at[sk_v.at[b * s + k]],
                                 rx_v.at[pl.ds(row, _CH)], sg[b])
                pltpu.async_copy(taby_h.at[sk_v.at[b * s + k]],
                                 ry_v.at[pl.ds(row, _CH)], sg[b])

        def wait_gathers(b):
            for k in range(s):
                row = (b * s + k) * _CH
                pltpu.make_async_copy(tabx_h.at[sk_v.at[b * s + k]],
                                      rx_v.at[pl.ds(row, _CH)], sg[b]).wait()
                pltpu.make_async_copy(taby_h.at[sk_v.at[b * s + k]],
                                      ry_v.at[pl.ds(row, _CH)], sg[b]).wait()

        def start_scatters(b):
            for k in range(s):
                row = (b * s + k) * _CH
                pltpu.async_copy(rx_v.at[pl.ds(row, _CH)],
                                 ax_sh.at[dst_v.at[b * s + k]], ss[b],
                                 add=True)
                pltpu.async_copy(ry_v.at[pl.ds(row, _CH)],
                                 ay_sh.at[dst_v.at[b * s + k]], ss[b],
                                 add=True)

        def wait_scatters(b):
            for k in range(s):
                row = (b * s + k) * _CH
                pltpu.make_async_copy(rx_v.at[pl.ds(row, _CH)],
                                      ax_sh.at[dst_v.at[b * s + k]],
                                      ss[b]).wait()
                pltpu.make_async_copy(ry_v.at[pl.ds(row, _CH)],
                                      ay_sh.at[dst_v.at[b * s + k]],
                                      ss[b]).wait()

        def scale(b):
            for k in range(s):
                rowbase = (b * s + k) * _CH
                wrow = w_v.at[b * s + k]

                @pl.loop(0, _CH)
                def _(ei):
                    idxv = jnp.full((16,), ei, _i32)
                    wb = plsc.load_gather(wrow, [idxv])
                    for j in range(h // 16):
                        slx = (rowbase + ei, pl.ds(j * 16, 16))
                        rx_v[slx] = rx_v[slx] * wb
                        ry_v[slx] = ry_v[slx] * wb

        load_group(0, 0)
        start_gathers(0)

        @pl.loop(0, ng // 3)
        def _(o3):
            for b in range(3):
                g = o3 * 3 + b
                b1 = (b + 1) % 3
                wait_gathers(b)
                if b == 2:
                    wait_scatters(b1)
                    @pl.when(o3 < ng // 3 - 1)
                    def _():
                        load_group(g + 1, b1)
                        start_gathers(b1)
                else:
                    @pl.when(o3 > 0)
                    def _():
                        wait_scatters(b1)
                    load_group(g + 1, b1)
                    start_gathers(b1)
                scale(b)
                start_scatters(b)

        wait_scatters(1)
        wait_scatters(2)
        plsc.subcore_barrier()
        pltpu.sync_copy(ax_sh.at[pl.ds(r0, psn)], ax_h.at[cid, pl.ds(r0, psn)])
        pltpu.sync_copy(ay_sh.at[pl.ds(r0, psn)], ay_h.at[cid, pl.ds(r0, psn)])

    kern = pl.kernel(
        body,
        out_type=[jax.ShapeDtypeStruct((_NC, n, h), _f32),
                  jax.ShapeDtypeStruct((_NC, n, h), _f32)],
        mesh=_mesh(),
        compiler_params=_SC_PARAMS,
        scratch_types=[
            pltpu.VMEM((3 * s, _CH), _i32),
            pltpu.VMEM((3 * s, _CH), _i32),
            pltpu.VMEM((3 * s, _CH), _f32),
            pltpu.VMEM((3 * s * _CH, h), _f32),
            pltpu.VMEM((3 * s * _CH, h), _f32),
            pltpu.VMEM_SHARED((n, h), _f32),
            pltpu.VMEM_SHARED((n, h), _f32),
            pltpu.SemaphoreType.DMA,
            pltpu.SemaphoreType.DMA,
            pltpu.SemaphoreType.DMA,
            pltpu.SemaphoreType.DMA,
            pltpu.SemaphoreType.DMA,
            pltpu.SemaphoreType.DMA,
        ],
    )
    zn = jnp.zeros((n, h), _f32)
    return kern(tabx, taby, sk2, w2, dst2, zn)


def _pairs(fin, feat, aa, bb, b):
    n, fd = fin.shape
    fr = feat.shape[1]
    nch = b // _CH

    def body(fin_h, feat_h, aa_h, bb_h, ddi_h, mol_h,
             ia_v, ib_v, fa_v, fb_v, ma_v, mb_v):
        cid = lax.axis_index("c")
        sid = lax.axis_index("s")
        wid = sid * _NC + cid

        @pl.loop(0, -(-nch // _NW))
        def _(i):
            chunk = wid + i * _NW

            @pl.when(chunk < nch)
            def _():
                base = chunk * _CH
                pltpu.sync_copy(aa_h.at[pl.ds(base, _CH)], ia_v.at[0])
                pltpu.sync_copy(bb_h.at[pl.ds(base, _CH)], ib_v.at[0])
                pltpu.sync_copy(fin_h.at[ia_v.at[0]], fa_v)
                pltpu.sync_copy(fin_h.at[ib_v.at[0]], fb_v)
                pltpu.sync_copy(feat_h.at[ia_v.at[0]], ma_v)
                pltpu.sync_copy(feat_h.at[ib_v.at[0]], mb_v)
                pltpu.sync_copy(fa_v, ddi_h.at[pl.ds(base, _CH), pl.ds(0, fd)])
                pltpu.sync_copy(fb_v, ddi_h.at[pl.ds(base, _CH), pl.ds(fd, fd)])
                pltpu.sync_copy(ma_v, mol_h.at[pl.ds(base, _CH), pl.ds(0, fr)])
                pltpu.sync_copy(mb_v, mol_h.at[pl.ds(base, _CH), pl.ds(fr, fr)])

    kern = pl.kernel(
        body,
        out_type=[jax.ShapeDtypeStruct((b, 2 * fd), _f32),
                  jax.ShapeDtypeStruct((b, 2 * fr), _f32)],
        mesh=_mesh(),
        compiler_params=_SC_PARAMS,
        scratch_types=[
            pltpu.VMEM((1, _CH), _i32),
            pltpu.VMEM((1, _CH), _i32),
            pltpu.VMEM((_CH, fd), _f32),
            pltpu.VMEM((_CH, fd), _f32),
            pltpu.VMEM((_CH, fr), _f32),
            pltpu.VMEM((_CH, fr), _f32),
        ],
    )
    return kern(fin, feat, aa, bb)


_PREC = lax.Precision.HIGHEST


def _dot(a, b):
    return jnp.dot(a, b, preferred_element_type=_f32, precision=_PREC)


def _mm_tab(x, wt, wr, br):
    n, k = x.shape
    ct = wt.shape[1]
    cr = wr.shape[1]

    def body(x_ref, wt_ref, wr_ref, tab_ref, rt_ref):
        xb = x_ref[...]
        tab_ref[...] = _dot(xb, wt_ref[...])
        rt_ref[...] = _dot(xb, wr_ref[...])

    return pl.pallas_call(
        body,
        grid=(n // br,),
        in_specs=[pl.BlockSpec((br, k), lambda i: (i, 0)),
                  pl.BlockSpec((k, ct), lambda i: (0, 0)),
                  pl.BlockSpec((k, cr), lambda i: (0, 0))],
        out_specs=[pl.BlockSpec((br, ct), lambda i: (i, 0)),
                   pl.BlockSpec((br, cr), lambda i: (i, 0))],
        out_shape=[jax.ShapeDtypeStruct((n, ct), _f32),
                   jax.ShapeDtypeStruct((n, cr), _f32)],
    )(x, wt, wr)


def _combine_mm(accp, rt, bvec, wt, wr, br):
    n, h = rt.shape
    ct = wt.shape[1]
    cr = wr.shape[1]

    def body(acc_ref, rt_ref, b_ref, wt_ref, wr_ref, x1_ref, tab_ref, rt2_ref):
        x1 = acc_ref[0] + acc_ref[1] + rt_ref[...] + b_ref[...]
        x1 = jnp.maximum(x1, 0.0)
        x1_ref[...] = x1
        tab_ref[...] = _dot(x1, wt_ref[...])
        rt2_ref[...] = _dot(x1, wr_ref[...])

    return pl.pallas_call(
        body,
        grid=(n // br,),
        in_specs=[pl.BlockSpec((_NC, br, h), lambda i: (0, i, 0)),
                  pl.BlockSpec((br, h), lambda i: (i, 0)),
                  pl.BlockSpec((1, h), lambda i: (0, 0)),
                  pl.BlockSpec((h, ct), lambda i: (0, 0)),
                  pl.BlockSpec((h, cr), lambda i: (0, 0))],
        out_specs=[pl.BlockSpec((br, h), lambda i: (i, 0)),
                   pl.BlockSpec((br, ct), lambda i: (i, 0)),
                   pl.BlockSpec((br, cr), lambda i: (i, 0))],
        out_shape=[jax.ShapeDtypeStruct((n, h), _f32),
                   jax.ShapeDtypeStruct((n, ct), _f32),
                   jax.ShapeDtypeStruct((n, cr), _f32)],
    )(accp, rt, bvec, wt, wr)


def _x2_combine(accp, rt2, bvec, br):
    n, h = rt2.shape
    nb = n // br

    def body(acc_ref, rt_ref, b_ref, x2_ref, ps_ref):
        x2 = acc_ref[0] + acc_ref[1] + rt_ref[...] + b_ref[...]
        x2_ref[...] = x2
        ps_ref[...] = jnp.sum(x2, axis=0, keepdims=True)[None]

    x2, ps = pl.pallas_call(
        body,
        grid=(nb,),
        in_specs=[pl.BlockSpec((_NC, br, h), lambda i: (0, i, 0)),
                  pl.BlockSpec((br, h), lambda i: (i, 0)),
                  pl.BlockSpec((1, h), lambda i: (0, 0))],
        out_specs=[pl.BlockSpec((br, h), lambda i: (i, 0)),
                   pl.BlockSpec((1, 1, h), lambda i: (i, 0, 0))],
        out_shape=[jax.ShapeDtypeStruct((n, h), _f32),
                   jax.ShapeDtypeStruct((nb, 1, h), _f32)],
    )(accp, rt2, bvec)
    return x2, ps.reshape(nb, h)


def _readout(psum, wd2t, n):
    nb, h = psum.shape

    def body(ps_ref, wd_ref, v_ref):
        tot = jnp.sum(ps_ref[...], axis=0, keepdims=True) * (1.0 / n)
        hvec = jax.nn.sigmoid(tot)
        v_ref[...] = _dot(hvec, wd_ref[...])

    return pl.pallas_call(
        body,
        grid=(1,),
        in_specs=[pl.BlockSpec((nb, h), lambda i: (0, 0)),
                  pl.BlockSpec((h, h), lambda i: (0, 0))],
        out_specs=[pl.BlockSpec((1, h), lambda i: (0, 0))],
        out_shape=[jax.ShapeDtypeStruct((1, h), _f32)],
    )(psum, wd2t)[0]


def _heads(x1o, x2o, x2oa, x2oaa, x2aa, v, bd, a0, a1, br):
    n, h1 = x1o.shape
    h2 = x2o.shape[1]

    def body(x1_ref, xo_ref, xoa_ref, xoaa_ref, xaa_ref, v_ref, bd_ref,
             a0_ref, a1_ref, ros_ref, roa_ref, rosa_ref, fin_ref):
        vv = v_ref[...]
        bdv = bd_ref[...]

        def mv(x):
            return jnp.sum(x * vv, axis=1, keepdims=True) + bdv

        xo = xo_ref[...]
        s_o = mv(xo)
        s_oa = mv(xoa_ref[...])
        s_oaa = mv(xoaa_ref[...])
        s_aa = mv(xaa_ref[...])
        ros_ref[...] = jnp.concatenate([s_o, s_oaa], axis=1)
        roa_ref[...] = jnp.concatenate([s_o, s_oa], axis=1)
        rosa_ref[...] = jnp.concatenate([s_o, s_aa], axis=1)
        fin_ref[...] = jnp.concatenate(
            [a0_ref[...] * x1_ref[...], a1_ref[...] * xo], axis=1)

    return pl.pallas_call(
        body,
        grid=(n // br,),
        in_specs=[pl.BlockSpec((br, h1), lambda i: (i, 0)),
                  pl.BlockSpec((br, h2), lambda i: (i, 0)),
                  pl.BlockSpec((br, h2), lambda i: (i, 0)),
                  pl.BlockSpec((br, h2), lambda i: (i, 0)),
                  pl.BlockSpec((br, h2), lambda i: (i, 0)),
                  pl.BlockSpec((1, h2), lambda i: (0, 0)),
                  pl.BlockSpec((1, 1), lambda i: (0, 0)),
                  pl.BlockSpec((1, 1), lambda i: (0, 0)),
                  pl.BlockSpec((1, 1), lambda i: (0, 0))],
        out_specs=[pl.BlockSpec((br, 2), lambda i: (i, 0)),
                   pl.BlockSpec((br, 2), lambda i: (i, 0)),
                   pl.BlockSpec((br, 2), lambda i: (i, 0)),
                   pl.BlockSpec((br, h1 + h2), lambda i: (i, 0))],
        out_shape=[jax.ShapeDtypeStruct((n, 2), _f32),
                   jax.ShapeDtypeStruct((n, 2), _f32),
                   jax.ShapeDtypeStruct((n, 2), _f32),
                   jax.ShapeDtypeStruct((n, h1 + h2), _f32)],
    )(x1o, x2o, x2oa, x2oaa, x2aa, v, bd, a0, a1)


def _mlp(ddi, l1, b1, l2, b2, l3, b3, br):
    b, d = ddi.shape
    c1 = l1.shape[1]
    c2 = l2.shape[1]
    c3 = l3.shape[1]

    def body(x_ref, l1_ref, b1_ref, l2_ref, b2_ref, l3_ref, b3_ref, o_ref):
        h = _dot(x_ref[...], l1_ref[...]) + b1_ref[...]
        h = jnp.where(h > 0, h, jnp.exp(h) - 1.0)
        h = _dot(h, l2_ref[...]) + b2_ref[...]
        h = jnp.where(h > 0, h, jnp.exp(h) - 1.0)
        o_ref[...] = _dot(h, l3_ref[...]) + b3_ref[...]

    return pl.pallas_call(
        body,
        grid=(b // br,),
        in_specs=[pl.BlockSpec((br, d), lambda i: (i, 0)),
                  pl.BlockSpec((d, c1), lambda i: (0, 0)),
                  pl.BlockSpec((1, c1), lambda i: (0, 0)),
                  pl.BlockSpec((c1, c2), lambda i: (0, 0)),
                  pl.BlockSpec((1, c2), lambda i: (0, 0)),
                  pl.BlockSpec((c2, c3), lambda i: (0, 0)),
                  pl.BlockSpec((1, c3), lambda i: (0, 0))],
        out_specs=[pl.BlockSpec((br, c3), lambda i: (i, 0))],
        out_shape=[jax.ShapeDtypeStruct((b, c3), _f32)],
    )(ddi, l1, b1, l2, b2, l3, b3)[0]


def kernel(x_o, x_a, edge_index, e_type, e_type1, idx, W1, root1, b1,
           W2, root2, b2, attt, Wd, bd, features1, L1, bl1, L2, bl2, L3, bl3):
    n, df = x_o.shape
    e = e_type.shape[0]
    r = W1.shape[0]
    h1 = W1.shape[2]
    h2 = W2.shape[2]
    b = idx.shape[1]
    src = edge_index[0]
    dst = edge_index[1]

    skey0, skey1, w0, w1 = _prep(src, dst, e_type, e_type1, n, r)

    ep = _NW * _CPW * _CH
    pad = ep - e
    zi = jnp.zeros((pad,), _i32)
    zf = jnp.zeros((pad,), _f32)
    skp0 = jnp.concatenate([skey0, zi]).reshape(-1, _CH)
    skp1 = jnp.concatenate([skey1, zi]).reshape(-1, _CH)
    wp0 = jnp.concatenate([w0, zf]).reshape(-1, _CH)
    wp1 = jnp.concatenate([w1, zf]).reshape(-1, _CH)
    dstp = jnp.concatenate([dst, zi]).reshape(-1, _CH)

    wt1 = jnp.transpose(W1, (1, 0, 2)).reshape(df, r * h1)
    tab_o, rt_o = _mm_tab(x_o, wt1, root1, 1000)
    tab_a, rt_a = _mm_tab(x_a, wt1, root1, 1000)

    tabv_o = tab_o.reshape(n * r, h1)
    tabv_a = tab_a.reshape(n * r, h1)
    acc_o0, acc_a0 = _conv_pair(tabv_o, tabv_a, skp0, wp0, dstp, n, h1, 1)
    acc_o1, acc_a1 = _conv_pair(tabv_o, tabv_a, skp1, wp1, dstp, n, h1, 1)

    wt2 = jnp.transpose(W2, (1, 0, 2)).reshape(h1, r * h2)
    b1r = b1.reshape(1, h1)
    x1_o, tab2_o, rt2_o = _combine_mm(acc_o0, rt_o, b1r, wt2, root2, 1000)
    x1_oa, tab2_oa, rt2_oa = _combine_mm(acc_a0, rt_a, b1r, wt2, root2, 1000)
    x1_oaa, tab2_oaa, rt2_oaa = _combine_mm(acc_o1, rt_o, b1r, wt2, root2, 1000)
    x1_aa, tab2_aa, rt2_aa = _combine_mm(acc_a1, rt_a, b1r, wt2, root2, 1000)

    acc2_o, acc2_oa = _conv_pair(tab2_o.reshape(n * r, h2),
                                 tab2_oa.reshape(n * r, h2),
                                 skp0, wp0, dstp, n, h2, 2)
    acc2_oaa, acc2_aa = _conv_pair(tab2_oaa.reshape(n * r, h2),
                                   tab2_aa.reshape(n * r, h2),
                                   skp1, wp1, dstp, n, h2, 2)

    b2r = b2.reshape(1, h2)
    x2_o, ps_o = _x2_combine(acc2_o, rt2_o, b2r, 1000)
    x2_oa, _ = _x2_combine(acc2_oa, rt2_oa, b2r, 1000)
    x2_oaa, _ = _x2_combine(acc2_oaa, rt2_oaa, b2r, 1000)
    x2_aa, _ = _x2_combine(acc2_aa, rt2_aa, b2r, 1000)

    v = _readout(ps_o, jnp.transpose(Wd[0]), n)
    ret_os, ret_oa, ret_os_a, final = _heads(
        x1_o, x2_o, x2_oa, x2_oaa, x2_aa, v,
        bd.reshape(1, 1), attt[0].reshape(1, 1), attt[1].reshape(1, 1), 1000)

    final_DDI, final_molecule = _pairs(final, features1, idx[0], idx[1], b)

    log = _mlp(final_DDI, L1, bl1.reshape(1, -1), L2, bl2.reshape(1, -1),
               L3, bl3.reshape(1, -1), 1024)

    return (log, ret_os, ret_oa, ret_os_a, x2_o, final_DDI, final_molecule,
            x2_oaa, x2_aa)
